# baseline (speedup 1.0000x reference)
import functools

import jax
import jax.numpy as jnp
from jax import lax
from jax.experimental import pallas as pl
from jax.experimental.pallas import tpu as pltpu
from jax.experimental.pallas import tpu_sc as plsc

F32 = jnp.float32

NU = 50000
NM = 10000
H = 128
E = 500000
EL = 100000

KE = 128
KI = 1024
NCHUNK = 3968
EPAD = 507904
NPAD = EPAD - E

NMP = 10240
RS = 12800
NRANGE = 4
ACCU = RS + 128

KD = 256
ELP = 106496
NDCH = ELP // KD

_MESH = plsc.VectorSubcoreMesh(
    core_axis_name="c", subcore_axis_name="s", num_cores=2, num_subcores=16)


def _zero_rows(ref, n, width):
  z = jnp.zeros((16,), F32)
  def body(i, _):
    for j in range(width // 16):
      ref[i, pl.ds(j * 16, 16)] = z
    return 0
  lax.fori_loop(0, n, body, 0)


def _fill_rows(ref, n, width, val):
  v = jnp.full((16,), val, F32)
  def body(i, _):
    for j in range(width // 16):
      ref[i, pl.ds(j * 16, 16)] = v
    return 0
  lax.fori_loop(0, n, body, 0)


def _sc_agg_movies(dst, table=None, src=None):
  gather = table is not None

  scratch = [
      pltpu.VMEM((KE,), jnp.int32),
      pltpu.VMEM((KE, H), F32),
      pltpu.VMEM_SHARED((NMP, H), F32),
  ]
  if gather:
    scratch += [pltpu.VMEM((KE,), jnp.int32), pltpu.SemaphoreType.DMA]

  @functools.partial(
      pl.kernel,
      out_type=jax.ShapeDtypeStruct((2 * NMP, H), F32),
      mesh=_MESH,
      scratch_types=scratch,
      name="sc_agg_movies" if gather else "sc_cnt_movies",
  )
  def k(*refs):
    if gather:
      t_hbm, src_hbm, dst_hbm, out_hbm, dstv, rows, acc, srcv, sem = refs
    else:
      dst_hbm, out_hbm, dstv, rows, acc = refs
    cid = lax.axis_index("c")
    sid = lax.axis_index("s")
    w = sid * 2 + cid
    _fill_rows(rows, KE, H, 0.0)
    r0 = sid * (NMP // 16)
    def zc(j, _):
      pltpu.sync_copy(rows, acc.at[pl.ds(r0 + j * KE, KE)])
      return 0
    lax.fori_loop(0, 5, zc, 0)
    plsc.subcore_barrier()
    if not gather:
      _fill_rows(rows, KE, H, 1.0)

    nper = NCHUNK // 32
    def body(c, _):
      base = (w * nper + c) * KE
      if gather:
        pltpu.sync_copy(src_hbm.at[pl.ds(base, KE)], srcv)
        cp = pltpu.async_copy(t_hbm.at[srcv], rows, sem)
        pltpu.sync_copy(dst_hbm.at[pl.ds(base, KE)], dstv)
        cp.wait()
      else:
        pltpu.sync_copy(dst_hbm.at[pl.ds(base, KE)], dstv)
      pltpu.sync_copy(rows, acc.at[dstv], add=True)
      return 0
    lax.fori_loop(0, nper, body, 0)
    plsc.subcore_barrier()

    o0 = cid * NMP + r0
    def wc(j, _):
      pltpu.sync_copy(acc.at[pl.ds(r0 + j * KE, KE)], rows)
      pltpu.sync_copy(rows, out_hbm.at[pl.ds(o0 + j * KE, KE)])
      return 0
    lax.fori_loop(0, 5, wc, 0)

  if gather:
    return k(table, src, dst)
  return k(dst)


KU = 64


def _sc_agg_users(dst, table=None, src=None):
  gather = table is not None

  if gather:
    scratch = [
        pltpu.VMEM((KU,), jnp.int32), pltpu.VMEM((KU,), jnp.int32),
        pltpu.VMEM((KU,), jnp.int32), pltpu.VMEM((KU,), jnp.int32),
        pltpu.VMEM((KU,), jnp.int32), pltpu.VMEM((KU,), jnp.int32),
        pltpu.VMEM((KU, H), F32), pltpu.VMEM((KU, H), F32),
        pltpu.VMEM_SHARED((ACCU, H), F32),
        pltpu.SemaphoreType.DMA, pltpu.SemaphoreType.DMA,
        pltpu.SemaphoreType.DMA, pltpu.SemaphoreType.DMA,
    ]
  else:
    scratch = [
        pltpu.VMEM((KU,), jnp.int32),
        pltpu.VMEM((KU,), jnp.int32),
        pltpu.VMEM((KU, H), F32),
        pltpu.VMEM_SHARED((ACCU, H), F32),
    ]

  @functools.partial(
      pl.kernel,
      out_type=jax.ShapeDtypeStruct((NRANGE * RS, H), F32),
      mesh=_MESH,
      scratch_types=scratch,
      name="sc_agg_users" if gather else "sc_cnt_users",
  )
  def k(*refs):
    if gather:
      (t_hbm, src_hbm, dst_hbm, out_hbm, srcv0, srcv1, dstv0, dstv1,
       idx20, idx21, rows0, rows1, acc, sg0, sg1, ss0, ss1) = refs
      srcvs, dstvs, idx2s = (srcv0, srcv1), (dstv0, dstv1), (idx20, idx21)
      rowss, sgs, sss = (rows0, rows1), (sg0, sg1), (ss0, ss1)
    else:
      dst_hbm, out_hbm, dstv0, idx20, rows0, acc = refs
    cid = lax.axis_index("c")
    sid = lax.axis_index("s")
    lane = lax.iota(jnp.int32, 16)
    nper = EPAD // KU // 16

    def compute_idx2(dstv, idx2, base_row):
      def ix(i, _):
        d = dstv[pl.ds(i * 16, 16)]
        lo = d - base_row
        m = (lo >= 0) & (lo < RS)
        idx2[pl.ds(i * 16, 16)] = jnp.where(m, lo, RS + lane)
        return 0
      lax.fori_loop(0, KU // 16, ix, 0)

    for rr in range(2):
      _fill_rows(rows0, KU, H, 0.0)
      z0 = sid * (ACCU // 16)
      def zc(j, _):
        pltpu.sync_copy(rows0, acc.at[pl.ds(z0 + j * KU, KU)])
        return 0
      lax.fori_loop(0, 12, zc, 0)
      pltpu.sync_copy(rows0.at[pl.ds(0, 40)], acc.at[pl.ds(z0 + 768, 40)])
      plsc.subcore_barrier()

      base_row = (cid * 2 + rr) * RS
      if gather:
        def step(c, b, first):
          base = (sid * nper + c) * KU
          if not first:
            pltpu.make_async_copy(rowss[b], acc.at[idx2s[b]], sss[b]).wait()
          pltpu.sync_copy(src_hbm.at[pl.ds(base, KU)], srcvs[b])
          cp = pltpu.async_copy(t_hbm.at[srcvs[b]], rowss[b], sgs[b])
          pltpu.sync_copy(dst_hbm.at[pl.ds(base, KU)], dstvs[b])
          compute_idx2(dstvs[b], idx2s[b], base_row)
          cp.wait()
          pltpu.async_copy(rowss[b], acc.at[idx2s[b]], sss[b], add=True)
        step(0, 0, True)
        step(1, 1, True)
        def body(cp2, _):
          step(cp2 * 2, 0, False)
          step(cp2 * 2 + 1, 1, False)
          return 0
        lax.fori_loop(1, nper // 2, body, 0)
        for b in range(2):
          pltpu.make_async_copy(rowss[b], acc.at[idx2s[b]], sss[b]).wait()
      else:
        _fill_rows(rows0, KU, H, 1.0)
        def body(c, _):
          base = (sid * nper + c) * KU
          pltpu.sync_copy(dst_hbm.at[pl.ds(base, KU)], dstv0)
          compute_idx2(dstv0, idx20, base_row)
          pltpu.sync_copy(rows0, acc.at[idx20], add=True)
          return 0
        lax.fori_loop(0, nper, body, 0)
      plsc.subcore_barrier()

      if not gather:
        _fill_rows(rows0, KU, H, 0.0)
      w0 = sid * (RS // 16)
      stage = rows0
      def wc(j, _):
        pltpu.sync_copy(acc.at[pl.ds(w0 + j * KU, KU)], stage)
        pltpu.sync_copy(stage, out_hbm.at[pl.ds(base_row + w0 + j * KU, KU)])
        return 0
      lax.fori_loop(0, 12, wc, 0)
      pltpu.sync_copy(acc.at[pl.ds(w0 + 768, 32)], stage.at[pl.ds(0, 32)])
      pltpu.sync_copy(stage.at[pl.ds(0, 32)],
                      out_hbm.at[pl.ds(base_row + w0 + 768, 32)])
      plsc.subcore_barrier()

  if gather:
    return k(table, src, dst)
  return k(dst)


def _sc_decoder(gu, gm, elr, elc):
  @functools.partial(
      pl.kernel,
      out_type=jax.ShapeDtypeStruct((ELP, H), F32),
      mesh=_MESH,
      scratch_types=[
          pltpu.VMEM((KD,), jnp.int32),
          pltpu.VMEM((KD,), jnp.int32),
          pltpu.VMEM((KD, H), F32),
          pltpu.VMEM((KD, H), F32),
          pltpu.SemaphoreType.DMA,
          pltpu.SemaphoreType.DMA,
      ],
      name="sc_decoder",
  )
  def k(gu_hbm, gm_hbm, elr_hbm, elc_hbm, out_hbm, rv, cv, a, b, sa, sb):
    cid = lax.axis_index("c")
    sid = lax.axis_index("s")
    w = sid * 2 + cid
    nper = NDCH // 32
    def body(c, _):
      base = (w * nper + c) * KD
      pltpu.sync_copy(elr_hbm.at[pl.ds(base, KD)], rv)
      cpa = pltpu.async_copy(gu_hbm.at[rv], a, sa)
      pltpu.sync_copy(elc_hbm.at[pl.ds(base, KD)], cv)
      cpb = pltpu.async_copy(gm_hbm.at[cv], b, sb)
      cpa.wait()
      cpb.wait()
      def add(i, _):
        for j in range(H // 16):
          sl = pl.ds(j * 16, 16)
          a[i, sl] = a[i, sl] + b[i, sl]
        return 0
      lax.fori_loop(0, KD, add, 0)
      pltpu.sync_copy(a, out_hbm.at[pl.ds(base, KD)])
      return 0
    lax.fori_loop(0, nper, body, 0)

  return k(gu, gm, elr, elc)


_WSPEC = pl.BlockSpec((H, H), lambda i: (0, 0))
_BSPEC = pl.BlockSpec((1, H), lambda i: (0, 0))


def _dot(a, b):
  return jnp.dot(a, b, preferred_element_type=F32)


def _rows_spec(rb):
  return pl.BlockSpec((rb, H), lambda i: (i, 0))


def _tc_transform2(emb, x, wl, wr, bl, rb):
  n = emb.shape[0]
  def body(e_ref, x_ref, wla, wlb, wra, wrb, b_ref, t_ref, s_ref):
    e = e_ref[...]
    xx = x_ref[...]
    t_ref[...] = _dot(e, wla[...]) + _dot(xx, wlb[...])
    s_ref[...] = _dot(e, wra[...]) + _dot(xx, wrb[...]) + b_ref[...]
  rs = _rows_spec(rb)
  return pl.pallas_call(
      body,
      grid=(n // rb,),
      in_specs=[rs, rs, _WSPEC, _WSPEC, _WSPEC, _WSPEC, _BSPEC],
      out_specs=[rs, rs],
      out_shape=[jax.ShapeDtypeStruct((n, H), F32)] * 2,
  )(emb, x, wl[:H], wl[H:], wr[:H], wr[H:], bl.reshape(1, H))


def _tc_transform1(h, wl, wr, bl, rb):
  n = h.shape[0]
  def body(h_ref, wl_ref, wr_ref, b_ref, t_ref, s_ref):
    hh = h_ref[...]
    t_ref[...] = _dot(hh, wl_ref[...])
    s_ref[...] = _dot(hh, wr_ref[...]) + b_ref[...]
  rs = _rows_spec(rb)
  return pl.pallas_call(
      body,
      grid=(n // rb,),
      in_specs=[rs, _WSPEC, _WSPEC, _BSPEC],
      out_specs=[rs, rs],
      out_shape=[jax.ShapeDtypeStruct((n, H), F32)] * 2,
  )(h, wl, wr, bl.reshape(1, H))


def _tc_combine(aggs, cnts, s, relu, rb):
  n = s.shape[0]
  na, nc = len(aggs), len(cnts)
  def body(*refs):
    agg_refs = refs[:na]
    cnt_refs = refs[na:na + nc]
    s_ref, o_ref = refs[na + nc], refs[na + nc + 1]
    a = agg_refs[0][...]
    for r in agg_refs[1:]:
      a = a + r[...]
    c = cnt_refs[0][...]
    for r in cnt_refs[1:]:
      c = c + r[...]
    v = a / jnp.maximum(c, 1.0) + s_ref[...]
    o_ref[...] = jnp.maximum(v, 0.0) if relu else v
  rs = _rows_spec(rb)
  return pl.pallas_call(
      body,
      grid=(n // rb,),
      in_specs=[rs] * (na + nc + 1),
      out_specs=rs,
      out_shape=jax.ShapeDtypeStruct((n, H), F32),
  )(*aggs, *cnts, s)


def _tc_dec_user(h, wx, w1a, rb):
  n = h.shape[0]
  def body(h_ref, wx_ref, w1_ref, o_ref):
    o_ref[...] = _dot(_dot(h_ref[...], wx_ref[...]), w1_ref[...])
  rs = _rows_spec(rb)
  return pl.pallas_call(
      body,
      grid=(n // rb,),
      in_specs=[rs, _WSPEC, _WSPEC],
      out_specs=rs,
      out_shape=jax.ShapeDtypeStruct((n, H), F32),
  )(h, wx, w1a)


def _tc_dec_movie(h, wx, w1b, w1a, bu, bi, b1, rb):
  n = h.shape[0]
  def body(h_ref, wx_ref, w1b_ref, w1a_ref, bu_ref, bi_ref, b1_ref, o_ref):
    c = (_dot(bu_ref[...], w1a_ref[...]) + _dot(bi_ref[...], w1b_ref[...])
         + b1_ref[...])
    o_ref[...] = _dot(_dot(h_ref[...], wx_ref[...]), w1b_ref[...]) + c
  rs = _rows_spec(rb)
  return pl.pallas_call(
      body,
      grid=(n // rb,),
      in_specs=[rs, _WSPEC, _WSPEC, _WSPEC, _BSPEC, _BSPEC, _BSPEC],
      out_specs=rs,
      out_shape=jax.ShapeDtypeStruct((n, H), F32),
  )(h, wx, w1b, w1a, bu.reshape(1, H), bi.reshape(1, H), b1.reshape(1, H))


def _tc_dec_final(z, w2row, b2row):
  zb = 1024
  def body(z_ref, w2_ref, b2_ref, o_ref):
    r = jnp.maximum(z_ref[...], 0.0)
    v = jnp.sum(r * w2_ref[...], axis=1)
    o_ref[...] = v.reshape(zb // H, H) + b2_ref[...][:, 0:1]
  return pl.pallas_call(
      body,
      grid=(ELP // zb,),
      in_specs=[pl.BlockSpec((zb, H), lambda i: (i, 0)), _BSPEC, _BSPEC],
      out_specs=pl.BlockSpec((zb // H, H), lambda i: (i, 0)),
      out_shape=jax.ShapeDtypeStruct((ELP // H, H), F32),
  )(z, w2row, b2row)


def kernel(user_id, movie_id, x_user, x_movie, um_src, um_dst,
           mu_src, mu_dst, el_row, el_col, params):
  p = params

  lane = jnp.arange(NPAD, dtype=jnp.int32) % 16
  zpad = jnp.zeros((NPAD,), jnp.int32)
  src_a = jnp.concatenate([um_src, zpad])
  dst_a = jnp.concatenate([um_dst, NM + lane])
  src_b = jnp.concatenate([um_dst, zpad])
  dst_b = jnp.concatenate([um_src, NU + lane])
  elpad = jnp.zeros((ELP - EL,), jnp.int32)
  elr = jnp.concatenate([el_row, elpad])
  elc = jnp.concatenate([el_col, elpad])

  cm2 = _sc_agg_movies(dst_a)
  cu = _sc_agg_users(dst_b)
  cnt_m = [cm2[:NM], cm2[NMP:NMP + NM]]
  cnt_u = [cu[:NU]]

  h_u = (p['user_emb'], x_user)
  h_m = (p['movie_emb'], x_movie)

  for l in (1, 2, 3):
    wl_um, bl_um, wr_um = p['Wl%d_um' % l], p['bl%d_um' % l], p['Wr%d_um' % l]
    wl_mu, bl_mu, wr_mu = p['Wl%d_mu' % l], p['bl%d_mu' % l], p['Wr%d_mu' % l]
    if l == 1:
      t_um, s_u = _tc_transform2(h_u[0], h_u[1], wl_um, wr_mu, bl_mu, 1000)
      t_mu, s_m = _tc_transform2(h_m[0], h_m[1], wl_mu, wr_um, bl_um, 1000)
    else:
      t_um, s_u = _tc_transform1(h_u, wl_um, wr_mu, bl_mu, 1000)
      t_mu, s_m = _tc_transform1(h_m, wl_mu, wr_um, bl_um, 1000)

    agg_m2 = _sc_agg_movies(dst_a, t_um, src_a)
    agg_u = _sc_agg_users(dst_b, t_mu, src_b)

    relu = l < 3
    h_m = _tc_combine([agg_m2[:NM], agg_m2[NMP:NMP + NM]], cnt_m, s_m,
                      relu, 1000)
    h_u = _tc_combine([agg_u[:NU]], cnt_u, s_u, relu, 1000)

  w1 = p['W1']
  gu = _tc_dec_user(h_u, p['Wu2i'], w1[:H], 1000)
  gm = _tc_dec_movie(h_m, p['Wi2u'], w1[H:], w1[:H],
                     p['bu2i'], p['bi2u'], p['b1'], 1000)

  z = _sc_decoder(gu, gm, elr, elc)

  w2row = p['W2'].reshape(1, H)
  b2row = jnp.broadcast_to(p['b2'].reshape(1, 1), (1, H))
  res = _tc_dec_final(z, w2row, b2row)
  return res.reshape(-1)[:EL]


# speedup vs baseline: 1.3574x; 1.3574x over previous
import functools

import jax
import jax.numpy as jnp
from jax import lax
from jax.experimental import pallas as pl
from jax.experimental.pallas import tpu as pltpu
from jax.experimental.pallas import tpu_sc as plsc

F32 = jnp.float32

NU = 50000
NM = 10000
H = 128
E = 500000
EL = 100000

KE = 128
NCHUNK = 3936
EPAD = NCHUNK * KE
NPAD = EPAD - E

NMP = 10240
RS = 12800
NRANGE = 4
ACCU = RS + 128

KD = 256
ELP = 106496
NDCH = ELP // KD

_MESH = plsc.VectorSubcoreMesh(
    core_axis_name="c", subcore_axis_name="s", num_cores=2, num_subcores=16)


def _zero_rows(ref, n, width):
  z = jnp.zeros((16,), F32)
  def body(i, _):
    for j in range(width // 16):
      ref[i, pl.ds(j * 16, 16)] = z
    return 0
  lax.fori_loop(0, n, body, 0)


def _fill_rows(ref, n, width, val):
  v = jnp.full((16,), val, F32)
  def body(i, _):
    for j in range(width // 16):
      ref[i, pl.ds(j * 16, 16)] = v
    return 0
  lax.fori_loop(0, n, body, 0)


def _sc_agg_movies(dst, table=None, src=None):
  gather = table is not None

  scratch = [
      pltpu.VMEM((KE,), jnp.int32),
      pltpu.VMEM((KE, H), F32),
      pltpu.VMEM_SHARED((NMP, H), F32),
  ]
  if gather:
    scratch += [pltpu.VMEM((KE,), jnp.int32), pltpu.SemaphoreType.DMA]

  @functools.partial(
      pl.kernel,
      out_type=jax.ShapeDtypeStruct((2 * NMP, H), F32),
      mesh=_MESH,
      scratch_types=scratch,
      name="sc_agg_movies" if gather else "sc_cnt_movies",
  )
  def k(*refs):
    if gather:
      t_hbm, src_hbm, dst_hbm, out_hbm, dstv, rows, acc, srcv, sem = refs
    else:
      dst_hbm, out_hbm, dstv, rows, acc = refs
    cid = lax.axis_index("c")
    sid = lax.axis_index("s")
    w = sid * 2 + cid
    _fill_rows(rows, KE, H, 0.0)
    r0 = sid * (NMP // 16)
    def zc(j, _):
      pltpu.sync_copy(rows, acc.at[pl.ds(r0 + j * KE, KE)])
      return 0
    lax.fori_loop(0, 5, zc, 0)
    plsc.subcore_barrier()
    if not gather:
      _fill_rows(rows, KE, H, 1.0)

    nper = NCHUNK // 32
    def body(c, _):
      base = (w * nper + c) * KE
      if gather:
        pltpu.sync_copy(src_hbm.at[pl.ds(base, KE)], srcv)
        cp = pltpu.async_copy(t_hbm.at[srcv], rows, sem)
        pltpu.sync_copy(dst_hbm.at[pl.ds(base, KE)], dstv)
        cp.wait()
      else:
        pltpu.sync_copy(dst_hbm.at[pl.ds(base, KE)], dstv)
      pltpu.sync_copy(rows, acc.at[dstv], add=True)
      return 0
    lax.fori_loop(0, nper, body, 0)
    plsc.subcore_barrier()

    o0 = cid * NMP + r0
    def wc(j, _):
      pltpu.sync_copy(acc.at[pl.ds(r0 + j * KE, KE)], rows)
      pltpu.sync_copy(rows, out_hbm.at[pl.ds(o0 + j * KE, KE)])
      return 0
    lax.fori_loop(0, 5, wc, 0)

  if gather:
    return k(table, src, dst)
  return k(dst)


def _sc_agg_users(dst, table=None, src=None):
  gather = table is not None

  scratch = [
      pltpu.VMEM((KE,), jnp.int32),
      pltpu.VMEM((KE,), jnp.int32),
      pltpu.VMEM((KE, H), F32),
      pltpu.VMEM_SHARED((ACCU, H), F32),
  ]
  if gather:
    scratch += [pltpu.VMEM((KE,), jnp.int32), pltpu.SemaphoreType.DMA]

  @functools.partial(
      pl.kernel,
      out_type=jax.ShapeDtypeStruct((NRANGE * RS, H), F32),
      mesh=_MESH,
      scratch_types=scratch,
      name="sc_agg_users" if gather else "sc_cnt_users",
  )
  def k(*refs):
    if gather:
      t_hbm, src_hbm, dst_hbm, out_hbm, dstv, idx2, rows, acc, srcv, sem = refs
    else:
      dst_hbm, out_hbm, dstv, idx2, rows, acc = refs
    cid = lax.axis_index("c")
    sid = lax.axis_index("s")
    lane = lax.iota(jnp.int32, 16)
    nper = NCHUNK // 16

    for rr in range(2):
      _fill_rows(rows, KE, H, 0.0)
      z0 = sid * (ACCU // 16)
      def zc(j, _):
        pltpu.sync_copy(rows, acc.at[pl.ds(z0 + j * KE, KE)])
        return 0
      lax.fori_loop(0, 6, zc, 0)
      pltpu.sync_copy(rows.at[pl.ds(0, 40)], acc.at[pl.ds(z0 + 768, 40)])
      plsc.subcore_barrier()
      if not gather:
        _fill_rows(rows, KE, H, 1.0)

      base_row = (cid * 2 + rr) * RS
      def body(c, _):
        base = (sid * nper + c) * KE
        if gather:
          pltpu.sync_copy(src_hbm.at[pl.ds(base, KE)], srcv)
          cp = pltpu.async_copy(t_hbm.at[srcv], rows, sem)
        pltpu.sync_copy(dst_hbm.at[pl.ds(base, KE)], dstv)
        def ix(i, _):
          d = dstv[pl.ds(i * 16, 16)]
          lo = d - base_row
          m = (lo >= 0) & (lo < RS)
          idx2[pl.ds(i * 16, 16)] = jnp.where(m, lo, RS + lane)
          return 0
        lax.fori_loop(0, KE // 16, ix, 0)
        if gather:
          cp.wait()
        pltpu.sync_copy(rows, acc.at[idx2], add=True)
        return 0
      lax.fori_loop(0, nper, body, 0)
      plsc.subcore_barrier()

      w0 = sid * (RS // 16)
      def wc(j, _):
        pltpu.sync_copy(acc.at[pl.ds(w0 + j * KE, KE)], rows)
        pltpu.sync_copy(rows, out_hbm.at[pl.ds(base_row + w0 + j * KE, KE)])
        return 0
      lax.fori_loop(0, 6, wc, 0)
      pltpu.sync_copy(acc.at[pl.ds(w0 + 768, 32)], rows.at[pl.ds(0, 32)])
      pltpu.sync_copy(rows.at[pl.ds(0, 32)],
                      out_hbm.at[pl.ds(base_row + w0 + 768, 32)])
      plsc.subcore_barrier()

  if gather:
    return k(table, src, dst)
  return k(dst)


def _sc_decoder(gu, gm, elr, elc):
  @functools.partial(
      pl.kernel,
      out_type=jax.ShapeDtypeStruct((ELP, H), F32),
      mesh=_MESH,
      scratch_types=[
          pltpu.VMEM((KD,), jnp.int32),
          pltpu.VMEM((KD,), jnp.int32),
          pltpu.VMEM((KD, H), F32),
          pltpu.VMEM((KD, H), F32),
          pltpu.SemaphoreType.DMA,
          pltpu.SemaphoreType.DMA,
      ],
      name="sc_decoder",
  )
  def k(gu_hbm, gm_hbm, elr_hbm, elc_hbm, out_hbm, rv, cv, a, b, sa, sb):
    cid = lax.axis_index("c")
    sid = lax.axis_index("s")
    w = sid * 2 + cid
    nper = NDCH // 32
    def body(c, _):
      base = (w * nper + c) * KD
      pltpu.sync_copy(elr_hbm.at[pl.ds(base, KD)], rv)
      cpa = pltpu.async_copy(gu_hbm.at[rv], a, sa)
      pltpu.sync_copy(elc_hbm.at[pl.ds(base, KD)], cv)
      cpb = pltpu.async_copy(gm_hbm.at[cv], b, sb)
      cpa.wait()
      cpb.wait()
      def add(i, _):
        for j in range(H // 16):
          sl = pl.ds(j * 16, 16)
          a[i, sl] = a[i, sl] + b[i, sl]
        return 0
      lax.fori_loop(0, KD, add, 0)
      pltpu.sync_copy(a, out_hbm.at[pl.ds(base, KD)])
      return 0
    lax.fori_loop(0, nper, body, 0)

  return k(gu, gm, elr, elc)


_WSPEC = pl.BlockSpec((H, H), lambda i: (0, 0))
_BSPEC = pl.BlockSpec((1, H), lambda i: (0, 0))


def _dot(a, b):
  return jnp.dot(a, b, preferred_element_type=F32)


def _rows_spec(rb):
  return pl.BlockSpec((rb, H), lambda i: (i, 0))


def _tc_transform2(emb, x, wl, wr, bl, rb):
  n = emb.shape[0]
  def body(e_ref, x_ref, wla, wlb, wra, wrb, b_ref, t_ref, s_ref):
    e = e_ref[...]
    xx = x_ref[...]
    t_ref[...] = _dot(e, wla[...]) + _dot(xx, wlb[...])
    s_ref[...] = _dot(e, wra[...]) + _dot(xx, wrb[...]) + b_ref[...]
  rs = _rows_spec(rb)
  return pl.pallas_call(
      body,
      grid=(n // rb,),
      in_specs=[rs, rs, _WSPEC, _WSPEC, _WSPEC, _WSPEC, _BSPEC],
      out_specs=[rs, rs],
      out_shape=[jax.ShapeDtypeStruct((n, H), F32)] * 2,
  )(emb, x, wl[:H], wl[H:], wr[:H], wr[H:], bl.reshape(1, H))


def _tc_transform1(h, wl, wr, bl, rb):
  n = h.shape[0]
  def body(h_ref, wl_ref, wr_ref, b_ref, t_ref, s_ref):
    hh = h_ref[...]
    t_ref[...] = _dot(hh, wl_ref[...])
    s_ref[...] = _dot(hh, wr_ref[...]) + b_ref[...]
  rs = _rows_spec(rb)
  return pl.pallas_call(
      body,
      grid=(n // rb,),
      in_specs=[rs, _WSPEC, _WSPEC, _BSPEC],
      out_specs=[rs, rs],
      out_shape=[jax.ShapeDtypeStruct((n, H), F32)] * 2,
  )(h, wl, wr, bl.reshape(1, H))


def _tc_combine(aggs, cnts, s, relu, rb):
  n = s.shape[0]
  na, nc = len(aggs), len(cnts)
  def body(*refs):
    agg_refs = refs[:na]
    cnt_refs = refs[na:na + nc]
    s_ref, o_ref = refs[na + nc], refs[na + nc + 1]
    a = agg_refs[0][...]
    for r in agg_refs[1:]:
      a = a + r[...]
    c = cnt_refs[0][...]
    for r in cnt_refs[1:]:
      c = c + r[...]
    v = a / jnp.maximum(c, 1.0) + s_ref[...]
    o_ref[...] = jnp.maximum(v, 0.0) if relu else v
  rs = _rows_spec(rb)
  return pl.pallas_call(
      body,
      grid=(n // rb,),
      in_specs=[rs] * (na + nc + 1),
      out_specs=rs,
      out_shape=jax.ShapeDtypeStruct((n, H), F32),
  )(*aggs, *cnts, s)


def _tc_dec_user(h, wx, w1a, rb):
  n = h.shape[0]
  def body(h_ref, wx_ref, w1_ref, o_ref):
    o_ref[...] = _dot(_dot(h_ref[...], wx_ref[...]), w1_ref[...])
  rs = _rows_spec(rb)
  return pl.pallas_call(
      body,
      grid=(n // rb,),
      in_specs=[rs, _WSPEC, _WSPEC],
      out_specs=rs,
      out_shape=jax.ShapeDtypeStruct((n, H), F32),
  )(h, wx, w1a)


def _tc_dec_movie(h, wx, w1b, w1a, bu, bi, b1, rb):
  n = h.shape[0]
  def body(h_ref, wx_ref, w1b_ref, w1a_ref, bu_ref, bi_ref, b1_ref, o_ref):
    c = (_dot(bu_ref[...], w1a_ref[...]) + _dot(bi_ref[...], w1b_ref[...])
         + b1_ref[...])
    o_ref[...] = _dot(_dot(h_ref[...], wx_ref[...]), w1b_ref[...]) + c
  rs = _rows_spec(rb)
  return pl.pallas_call(
      body,
      grid=(n // rb,),
      in_specs=[rs, _WSPEC, _WSPEC, _WSPEC, _BSPEC, _BSPEC, _BSPEC],
      out_specs=rs,
      out_shape=jax.ShapeDtypeStruct((n, H), F32),
  )(h, wx, w1b, w1a, bu.reshape(1, H), bi.reshape(1, H), b1.reshape(1, H))


def _tc_dec_final(z, w2row, b2row):
  zb = 1024
  def body(z_ref, w2_ref, b2_ref, o_ref):
    r = jnp.maximum(z_ref[...], 0.0)
    v = jnp.sum(r * w2_ref[...], axis=1)
    o_ref[...] = v.reshape(zb // H, H) + b2_ref[...][:, 0:1]
  return pl.pallas_call(
      body,
      grid=(ELP // zb,),
      in_specs=[pl.BlockSpec((zb, H), lambda i: (i, 0)), _BSPEC, _BSPEC],
      out_specs=pl.BlockSpec((zb // H, H), lambda i: (i, 0)),
      out_shape=jax.ShapeDtypeStruct((ELP // H, H), F32),
  )(z, w2row, b2row)


def kernel(user_id, movie_id, x_user, x_movie, um_src, um_dst,
           mu_src, mu_dst, el_row, el_col, params):
  p = params

  lane = jnp.arange(NPAD, dtype=jnp.int32) % 16
  zpad = jnp.zeros((NPAD,), jnp.int32)
  src_a = jnp.concatenate([um_src, zpad])
  dst_a = jnp.concatenate([um_dst, NM + lane])
  src_b = jnp.concatenate([um_dst, zpad])
  dst_b = jnp.concatenate([um_src, NU + lane])
  elpad = jnp.zeros((ELP - EL,), jnp.int32)
  elr = jnp.concatenate([el_row, elpad])
  elc = jnp.concatenate([el_col, elpad])

  cm2 = _sc_agg_movies(dst_a)
  cu = _sc_agg_users(dst_b)
  cnt_m = [cm2[:NM], cm2[NMP:NMP + NM]]
  cnt_u = [cu[:NU]]

  h_u = (p['user_emb'], x_user)
  h_m = (p['movie_emb'], x_movie)

  for l in (1, 2, 3):
    wl_um, bl_um, wr_um = p['Wl%d_um' % l], p['bl%d_um' % l], p['Wr%d_um' % l]
    wl_mu, bl_mu, wr_mu = p['Wl%d_mu' % l], p['bl%d_mu' % l], p['Wr%d_mu' % l]
    if l == 1:
      t_um, s_u = _tc_transform2(h_u[0], h_u[1], wl_um, wr_mu, bl_mu, 1000)
      t_mu, s_m = _tc_transform2(h_m[0], h_m[1], wl_mu, wr_um, bl_um, 1000)
    else:
      t_um, s_u = _tc_transform1(h_u, wl_um, wr_mu, bl_mu, 1000)
      t_mu, s_m = _tc_transform1(h_m, wl_mu, wr_um, bl_um, 1000)

    agg_m2 = _sc_agg_movies(dst_a, t_um, src_a)
    agg_u = _sc_agg_users(dst_b, t_mu, src_b)

    relu = l < 3
    h_m = _tc_combine([agg_m2[:NM], agg_m2[NMP:NMP + NM]], cnt_m, s_m,
                      relu, 1000)
    h_u = _tc_combine([agg_u[:NU]], cnt_u, s_u, relu, 1000)

  w1 = p['W1']
  gu = _tc_dec_user(h_u, p['Wu2i'], w1[:H], 1000)
  gm = _tc_dec_movie(h_m, p['Wi2u'], w1[H:], w1[:H],
                     p['bu2i'], p['bi2u'], p['b1'], 1000)

  z = _sc_decoder(gu, gm, elr, elc)

  w2row = p['W2'].reshape(1, H)
  b2row = jnp.broadcast_to(p['b2'].reshape(1, 1), (1, H))
  res = _tc_dec_final(z, w2row, b2row)
  return res.reshape(-1)[:EL]


# speedup vs baseline: 1.3965x; 1.0288x over previous
import functools

import jax
import jax.numpy as jnp
from jax import lax
from jax.experimental import pallas as pl
from jax.experimental.pallas import tpu as pltpu
from jax.experimental.pallas import tpu_sc as plsc

F32 = jnp.float32

NU = 50000
NM = 10000
H = 128
E = 500000
EL = 100000

KE = 128
NCHUNK = 3936
EPAD = NCHUNK * KE
NPAD = EPAD - E

NMP = 10240
RS = 12800
NRANGE = 4
ACCU = RS + 128

KD = 256
ELP = 106496
NDCH = ELP // KD

_MESH = plsc.VectorSubcoreMesh(
    core_axis_name="c", subcore_axis_name="s", num_cores=2, num_subcores=16)


def _fill_rows(ref, n, width, val):
  v = jnp.full((16,), val, F32)
  def body(i, _):
    for j in range(width // 16):
      ref[i, pl.ds(j * 16, 16)] = v
    return 0
  lax.fori_loop(0, n, body, 0)


def _sc_agg_movies(dst, table=None, src=None):
  gather = table is not None

  if gather:
    scratch = [
        pltpu.VMEM((KE,), jnp.int32), pltpu.VMEM((KE,), jnp.int32),
        pltpu.VMEM((KE,), jnp.int32), pltpu.VMEM((KE,), jnp.int32),
        pltpu.VMEM((KE, H), F32), pltpu.VMEM((KE, H), F32),
        pltpu.VMEM_SHARED((NMP, H), F32),
        pltpu.SemaphoreType.DMA, pltpu.SemaphoreType.DMA,
        pltpu.SemaphoreType.DMA, pltpu.SemaphoreType.DMA,
    ]
  else:
    scratch = [
        pltpu.VMEM((KE,), jnp.int32),
        pltpu.VMEM((KE, H), F32),
        pltpu.VMEM_SHARED((NMP, H), F32),
    ]

  @functools.partial(
      pl.kernel,
      out_type=jax.ShapeDtypeStruct((2 * NMP, H), F32),
      mesh=_MESH,
      scratch_types=scratch,
      name="sc_agg_movies" if gather else "sc_cnt_movies",
  )
  def k(*refs):
    if gather:
      (t_hbm, src_hbm, dst_hbm, out_hbm, srcv0, srcv1, dstv0, dstv1,
       rows0, rows1, acc, sg0, sg1, ss0, ss1) = refs
      srcvs, dstvs = (srcv0, srcv1), (dstv0, dstv1)
      rowss, sgs, sss = (rows0, rows1), (sg0, sg1), (ss0, ss1)
    else:
      dst_hbm, out_hbm, dstv0, rows0, acc = refs
    cid = lax.axis_index("c")
    sid = lax.axis_index("s")
    w = sid * 2 + cid
    _fill_rows(rows0, KE, H, 0.0)
    r0 = sid * (NMP // 16)
    def zc(j, _):
      pltpu.sync_copy(rows0, acc.at[pl.ds(r0 + j * KE, KE)])
      return 0
    lax.fori_loop(0, 5, zc, 0)
    plsc.subcore_barrier()

    nper = NCHUNK // 32
    if gather:
      def step(c, b, first):
        base = (w * nper + c) * KE
        if not first:
          pltpu.make_async_copy(rowss[b], acc.at[dstvs[b]], sss[b]).wait()
        pltpu.sync_copy(src_hbm.at[pl.ds(base, KE)], srcvs[b])
        cp = pltpu.async_copy(t_hbm.at[srcvs[b]], rowss[b], sgs[b])
        pltpu.sync_copy(dst_hbm.at[pl.ds(base, KE)], dstvs[b])
        cp.wait()
        pltpu.async_copy(rowss[b], acc.at[dstvs[b]], sss[b], add=True)
      step(0, 0, True)
      step(1, 1, True)
      def body(cp2, _):
        step(cp2 * 2, 0, False)
        step(cp2 * 2 + 1, 1, False)
        return 0
      lax.fori_loop(1, nper // 2, body, 0)
      step(nper - 1, 0, False)
      for b in range(2):
        pltpu.make_async_copy(rowss[b], acc.at[dstvs[b]], sss[b]).wait()
    else:
      _fill_rows(rows0, KE, H, 1.0)
      def body(c, _):
        base = (w * nper + c) * KE
        pltpu.sync_copy(dst_hbm.at[pl.ds(base, KE)], dstv0)
        pltpu.sync_copy(rows0, acc.at[dstv0], add=True)
        return 0
      lax.fori_loop(0, nper, body, 0)
    plsc.subcore_barrier()

    o0 = cid * NMP + r0
    stage = rows0
    def wc(j, _):
      pltpu.sync_copy(acc.at[pl.ds(r0 + j * KE, KE)], stage)
      pltpu.sync_copy(stage, out_hbm.at[pl.ds(o0 + j * KE, KE)])
      return 0
    lax.fori_loop(0, 5, wc, 0)

  if gather:
    return k(table, src, dst)
  return k(dst)


def _sc_agg_users(dst, table=None, src=None):
  gather = table is not None

  scratch = [
      pltpu.VMEM((KE,), jnp.int32),
      pltpu.VMEM((KE,), jnp.int32),
      pltpu.VMEM((KE, H), F32),
      pltpu.VMEM_SHARED((ACCU, H), F32),
  ]
  if gather:
    scratch += [pltpu.VMEM((KE,), jnp.int32), pltpu.SemaphoreType.DMA]

  @functools.partial(
      pl.kernel,
      out_type=jax.ShapeDtypeStruct((NRANGE * RS, H), F32),
      mesh=_MESH,
      scratch_types=scratch,
      name="sc_agg_users" if gather else "sc_cnt_users",
  )
  def k(*refs):
    if gather:
      t_hbm, src_hbm, dst_hbm, out_hbm, dstv, idx2, rows, acc, srcv, sem = refs
    else:
      dst_hbm, out_hbm, dstv, idx2, rows, acc = refs
    cid = lax.axis_index("c")
    sid = lax.axis_index("s")
    lane = lax.iota(jnp.int32, 16)
    nper = NCHUNK // 16

    for rr in range(2):
      _fill_rows(rows, KE, H, 0.0)
      z0 = sid * (ACCU // 16)
      def zc(j, _):
        pltpu.sync_copy(rows, acc.at[pl.ds(z0 + j * KE, KE)])
        return 0
      lax.fori_loop(0, 6, zc, 0)
      pltpu.sync_copy(rows.at[pl.ds(0, 40)], acc.at[pl.ds(z0 + 768, 40)])
      plsc.subcore_barrier()
      if not gather:
        _fill_rows(rows, KE, H, 1.0)

      base_row = (cid * 2 + rr) * RS
      def body(c, _):
        base = (sid * nper + c) * KE
        if gather:
          pltpu.sync_copy(src_hbm.at[pl.ds(base, KE)], srcv)
          cp = pltpu.async_copy(t_hbm.at[srcv], rows, sem)
        pltpu.sync_copy(dst_hbm.at[pl.ds(base, KE)], dstv)
        def ix(i, _):
          d = dstv[pl.ds(i * 16, 16)]
          lo = d - base_row
          m = (lo >= 0) & (lo < RS)
          idx2[pl.ds(i * 16, 16)] = jnp.where(m, lo, RS + lane)
          return 0
        lax.fori_loop(0, KE // 16, ix, 0)
        if gather:
          cp.wait()
        pltpu.sync_copy(rows, acc.at[idx2], add=True)
        return 0
      lax.fori_loop(0, nper, body, 0)
      plsc.subcore_barrier()

      w0 = sid * (RS // 16)
      def wc(j, _):
        pltpu.sync_copy(acc.at[pl.ds(w0 + j * KE, KE)], rows)
        pltpu.sync_copy(rows, out_hbm.at[pl.ds(base_row + w0 + j * KE, KE)])
        return 0
      lax.fori_loop(0, 6, wc, 0)
      pltpu.sync_copy(acc.at[pl.ds(w0 + 768, 32)], rows.at[pl.ds(0, 32)])
      pltpu.sync_copy(rows.at[pl.ds(0, 32)],
                      out_hbm.at[pl.ds(base_row + w0 + 768, 32)])
      plsc.subcore_barrier()

  if gather:
    return k(table, src, dst)
  return k(dst)


def _sc_decoder(gu, gm, elr, elc):
  @functools.partial(
      pl.kernel,
      out_type=jax.ShapeDtypeStruct((ELP, H), F32),
      mesh=_MESH,
      scratch_types=[
          pltpu.VMEM((KD,), jnp.int32),
          pltpu.VMEM((KD,), jnp.int32),
          pltpu.VMEM((KD, H), F32),
          pltpu.VMEM((KD, H), F32),
          pltpu.SemaphoreType.DMA,
          pltpu.SemaphoreType.DMA,
      ],
      name="sc_decoder",
  )
  def k(gu_hbm, gm_hbm, elr_hbm, elc_hbm, out_hbm, rv, cv, a, b, sa, sb):
    cid = lax.axis_index("c")
    sid = lax.axis_index("s")
    w = sid * 2 + cid
    nper = NDCH // 32
    def body(c, _):
      base = (w * nper + c) * KD
      pltpu.sync_copy(elr_hbm.at[pl.ds(base, KD)], rv)
      cpa = pltpu.async_copy(gu_hbm.at[rv], a, sa)
      pltpu.sync_copy(elc_hbm.at[pl.ds(base, KD)], cv)
      cpb = pltpu.async_copy(gm_hbm.at[cv], b, sb)
      cpa.wait()
      cpb.wait()
      def add(i, _):
        for j in range(H // 16):
          sl = pl.ds(j * 16, 16)
          a[i, sl] = a[i, sl] + b[i, sl]
        return 0
      lax.fori_loop(0, KD, add, 0)
      pltpu.sync_copy(a, out_hbm.at[pl.ds(base, KD)])
      return 0
    lax.fori_loop(0, nper, body, 0)

  return k(gu, gm, elr, elc)


_WSPEC = pl.BlockSpec((H, H), lambda i: (0, 0))
_BSPEC = pl.BlockSpec((1, H), lambda i: (0, 0))


def _dot(a, b):
  return jnp.dot(a, b, preferred_element_type=F32)


def _rows_spec(rb):
  return pl.BlockSpec((rb, H), lambda i: (i, 0))


def _tc_transform2(emb, x, wl, wr, bl, rb):
  n = emb.shape[0]
  def body(e_ref, x_ref, wla, wlb, wra, wrb, b_ref, t_ref, s_ref):
    e = e_ref[...]
    xx = x_ref[...]
    t_ref[...] = _dot(e, wla[...]) + _dot(xx, wlb[...])
    s_ref[...] = _dot(e, wra[...]) + _dot(xx, wrb[...]) + b_ref[...]
  rs = _rows_spec(rb)
  return pl.pallas_call(
      body,
      grid=(n // rb,),
      in_specs=[rs, rs, _WSPEC, _WSPEC, _WSPEC, _WSPEC, _BSPEC],
      out_specs=[rs, rs],
      out_shape=[jax.ShapeDtypeStruct((n, H), F32)] * 2,
  )(emb, x, wl[:H], wl[H:], wr[:H], wr[H:], bl.reshape(1, H))


def _tc_transform1(h, wl, wr, bl, rb):
  n = h.shape[0]
  def body(h_ref, wl_ref, wr_ref, b_ref, t_ref, s_ref):
    hh = h_ref[...]
    t_ref[...] = _dot(hh, wl_ref[...])
    s_ref[...] = _dot(hh, wr_ref[...]) + b_ref[...]
  rs = _rows_spec(rb)
  return pl.pallas_call(
      body,
      grid=(n // rb,),
      in_specs=[rs, _WSPEC, _WSPEC, _BSPEC],
      out_specs=[rs, rs],
      out_shape=[jax.ShapeDtypeStruct((n, H), F32)] * 2,
  )(h, wl, wr, bl.reshape(1, H))


def _tc_combine(aggs, cnts, s, relu, rb):
  n = s.shape[0]
  na, nc = len(aggs), len(cnts)
  def body(*refs):
    agg_refs = refs[:na]
    cnt_refs = refs[na:na + nc]
    s_ref, o_ref = refs[na + nc], refs[na + nc + 1]
    a = agg_refs[0][...]
    for r in agg_refs[1:]:
      a = a + r[...]
    c = cnt_refs[0][...]
    for r in cnt_refs[1:]:
      c = c + r[...]
    v = a / jnp.maximum(c, 1.0) + s_ref[...]
    o_ref[...] = jnp.maximum(v, 0.0) if relu else v
  rs = _rows_spec(rb)
  return pl.pallas_call(
      body,
      grid=(n // rb,),
      in_specs=[rs] * (na + nc + 1),
      out_specs=rs,
      out_shape=jax.ShapeDtypeStruct((n, H), F32),
  )(*aggs, *cnts, s)


def _tc_dec_user(h, wx, w1a, rb):
  n = h.shape[0]
  def body(h_ref, wx_ref, w1_ref, o_ref):
    o_ref[...] = _dot(_dot(h_ref[...], wx_ref[...]), w1_ref[...])
  rs = _rows_spec(rb)
  return pl.pallas_call(
      body,
      grid=(n // rb,),
      in_specs=[rs, _WSPEC, _WSPEC],
      out_specs=rs,
      out_shape=jax.ShapeDtypeStruct((n, H), F32),
  )(h, wx, w1a)


def _tc_dec_movie(h, wx, w1b, w1a, bu, bi, b1, rb):
  n = h.shape[0]
  def body(h_ref, wx_ref, w1b_ref, w1a_ref, bu_ref, bi_ref, b1_ref, o_ref):
    c = (_dot(bu_ref[...], w1a_ref[...]) + _dot(bi_ref[...], w1b_ref[...])
         + b1_ref[...])
    o_ref[...] = _dot(_dot(h_ref[...], wx_ref[...]), w1b_ref[...]) + c
  rs = _rows_spec(rb)
  return pl.pallas_call(
      body,
      grid=(n // rb,),
      in_specs=[rs, _WSPEC, _WSPEC, _WSPEC, _BSPEC, _BSPEC, _BSPEC],
      out_specs=rs,
      out_shape=jax.ShapeDtypeStruct((n, H), F32),
  )(h, wx, w1b, w1a, bu.reshape(1, H), bi.reshape(1, H), b1.reshape(1, H))


def _tc_dec_final(z, w2row, b2row):
  zb = 1024
  def body(z_ref, w2_ref, b2_ref, o_ref):
    r = jnp.maximum(z_ref[...], 0.0)
    v = jnp.sum(r * w2_ref[...], axis=1)
    o_ref[...] = v.reshape(zb // H, H) + b2_ref[...][:, 0:1]
  return pl.pallas_call(
      body,
      grid=(ELP // zb,),
      in_specs=[pl.BlockSpec((zb, H), lambda i: (i, 0)), _BSPEC, _BSPEC],
      out_specs=pl.BlockSpec((zb // H, H), lambda i: (i, 0)),
      out_shape=jax.ShapeDtypeStruct((ELP // H, H), F32),
  )(z, w2row, b2row)


def kernel(user_id, movie_id, x_user, x_movie, um_src, um_dst,
           mu_src, mu_dst, el_row, el_col, params):
  p = params

  lane = jnp.arange(NPAD, dtype=jnp.int32) % 16
  zpad = jnp.zeros((NPAD,), jnp.int32)
  src_a = jnp.concatenate([um_src, zpad])
  dst_a = jnp.concatenate([um_dst, NM + lane])
  src_b = jnp.concatenate([um_dst, zpad])
  dst_b = jnp.concatenate([um_src, NU + lane])
  elpad = jnp.zeros((ELP - EL,), jnp.int32)
  elr = jnp.concatenate([el_row, elpad])
  elc = jnp.concatenate([el_col, elpad])

  cm2 = _sc_agg_movies(dst_a)
  cu = _sc_agg_users(dst_b)
  cnt_m = [cm2[:NM], cm2[NMP:NMP + NM]]
  cnt_u = [cu[:NU]]

  h_u = (p['user_emb'], x_user)
  h_m = (p['movie_emb'], x_movie)

  for l in (1, 2, 3):
    wl_um, bl_um, wr_um = p['Wl%d_um' % l], p['bl%d_um' % l], p['Wr%d_um' % l]
    wl_mu, bl_mu, wr_mu = p['Wl%d_mu' % l], p['bl%d_mu' % l], p['Wr%d_mu' % l]
    if l == 1:
      t_um, s_u = _tc_transform2(h_u[0], h_u[1], wl_um, wr_mu, bl_mu, 1000)
      t_mu, s_m = _tc_transform2(h_m[0], h_m[1], wl_mu, wr_um, bl_um, 1000)
    else:
      t_um, s_u = _tc_transform1(h_u, wl_um, wr_mu, bl_mu, 1000)
      t_mu, s_m = _tc_transform1(h_m, wl_mu, wr_um, bl_um, 1000)

    agg_m2 = _sc_agg_movies(dst_a, t_um, src_a)
    agg_u = _sc_agg_users(dst_b, t_mu, src_b)

    relu = l < 3
    h_m = _tc_combine([agg_m2[:NM], agg_m2[NMP:NMP + NM]], cnt_m, s_m,
                      relu, 1000)
    h_u = _tc_combine([agg_u[:NU]], cnt_u, s_u, relu, 1000)

  w1 = p['W1']
  gu = _tc_dec_user(h_u, p['Wu2i'], w1[:H], 1000)
  gm = _tc_dec_movie(h_m, p['Wi2u'], w1[H:], w1[:H],
                     p['bu2i'], p['bi2u'], p['b1'], 1000)

  z = _sc_decoder(gu, gm, elr, elc)

  w2row = p['W2'].reshape(1, H)
  b2row = jnp.broadcast_to(p['b2'].reshape(1, 1), (1, H))
  res = _tc_dec_final(z, w2row, b2row)
  return res.reshape(-1)[:EL]
